# Initial kernel scaffold; baseline (speedup 1.0000x reference)
import jax
import jax.numpy as jnp
from jax.experimental import pallas as pl


def kernel(x, Wq, Wk, Wv, Wr, Wo):
    raise NotImplementedError("write your pallas kernel here")


# speedup vs baseline: 1.4256x; 1.4256x over previous
import functools
import math

import jax
import jax.numpy as jnp
from jax.experimental import pallas as pl

EMBED_DIM = 768
NUM_HEADS = 12
HEAD_DIM = EMBED_DIM // NUM_HEADS
NUM_NODES = 8
ROPE_BASE = 10000.0

QBLK = 256


def _proj_route_kernel(x_ref, wq_ref, wk_ref, wv_ref, wr_ref,
                       q_out, k_out, v_out, node_out):
    x = x_ref[...]
    n = x.shape[0]
    hd = wq_ref.shape[1]
    K = wr_ref.shape[1]
    f32 = jnp.float32

    q = jax.lax.dot_general(x, wq_ref[0], (((1,), (1,)), ((), ())),
                            preferred_element_type=f32)
    k = jax.lax.dot_general(x, wk_ref[0], (((1,), (1,)), ((), ())),
                            preferred_element_type=f32)
    v = jax.lax.dot_general(x, wv_ref[0], (((1,), (1,)), ((), ())),
                            preferred_element_type=f32)
    logits = jax.lax.dot_general(x, wr_ref[0], (((1,), (1,)), ((), ())),
                                 preferred_element_type=f32)

    idx = jax.lax.broadcasted_iota(jnp.int32, (n, K), 1)
    mx = jnp.max(logits, axis=1, keepdims=True)
    node = jnp.min(jnp.where(logits == mx, idx, K), axis=1, keepdims=True)

    onehot = (idx == node).astype(f32)
    cum = onehot
    shift = 1
    while shift < n:
        zeros = jnp.zeros((shift, K), dtype=f32)
        cum = cum + jnp.concatenate([zeros, cum[:-shift, :]], axis=0)
        shift *= 2
    pos = jnp.sum(onehot * cum, axis=1, keepdims=True) - 1.0

    half = hd // 2
    i2 = jax.lax.broadcasted_iota(jnp.int32, (1, half), 1).astype(f32)
    inv_freq = jnp.exp(i2 * (-2.0 * math.log(ROPE_BASE) / hd))
    ang = pos * inv_freq
    cos = jnp.cos(ang)
    sin = jnp.sin(ang)
    cos2 = jnp.concatenate([cos, cos], axis=1)
    sin2 = jnp.concatenate([sin, sin], axis=1)

    def rot_half(u):
        return jnp.concatenate([-u[:, half:], u[:, :half]], axis=1)

    q_out[0] = q * cos2 + rot_half(q) * sin2
    k_out[0] = k * cos2 + rot_half(k) * sin2
    v_out[0] = v
    node_out[0] = node


def _attn_kernel(q_ref, k_ref, v_ref, nc_ref, nr_ref, o_ref):
    qi = pl.program_id(1)
    q = q_ref[0]
    k = k_ref[0]
    v = v_ref[0]
    n = k.shape[0]
    hd = q.shape[1]
    scale = hd ** -0.5

    s = jax.lax.dot_general(q, k, (((1,), (1,)), ((), ())),
                            preferred_element_type=jnp.float32) * scale
    row = jax.lax.broadcasted_iota(jnp.int32, (QBLK, n), 0) + qi * QBLK
    col = jax.lax.broadcasted_iota(jnp.int32, (QBLK, n), 1)
    same = nc_ref[0] == nr_ref[0]
    mask = same & (row >= col)
    s = jnp.where(mask, s, jnp.float32(-1e9))
    m = jnp.max(s, axis=1, keepdims=True)
    e = jnp.exp(s - m)
    p = e / jnp.sum(e, axis=1, keepdims=True)
    out = jax.lax.dot_general(p, v, (((1,), (0,)), ((), ())),
                              preferred_element_type=jnp.float32)
    o_ref[...] = out[None]


def _outproj_kernel(y_ref, wo_ref, o_ref):
    h = pl.program_id(0)
    part = jax.lax.dot_general(y_ref[0], wo_ref[0], (((1,), (1,)), ((), ())),
                               preferred_element_type=jnp.float32)

    @pl.when(h == 0)
    def _():
        o_ref[...] = part

    @pl.when(h != 0)
    def _():
        o_ref[...] += part


@jax.jit
def kernel(x, Wq, Wk, Wv, Wr, Wo):
    B, N, D = x.shape
    H, hd, K = NUM_HEADS, HEAD_DIM, NUM_NODES
    x2 = x.reshape(N, D)

    q, k, v, node = pl.pallas_call(
        _proj_route_kernel,
        grid=(H,),
        in_specs=[
            pl.BlockSpec((N, D), lambda h: (0, 0)),
            pl.BlockSpec((1, hd, D), lambda h: (h, 0, 0)),
            pl.BlockSpec((1, hd, D), lambda h: (h, 0, 0)),
            pl.BlockSpec((1, hd, D), lambda h: (h, 0, 0)),
            pl.BlockSpec((1, K, D), lambda h: (h, 0, 0)),
        ],
        out_specs=[
            pl.BlockSpec((1, N, hd), lambda h: (h, 0, 0)),
            pl.BlockSpec((1, N, hd), lambda h: (h, 0, 0)),
            pl.BlockSpec((1, N, hd), lambda h: (h, 0, 0)),
            pl.BlockSpec((1, N, 1), lambda h: (h, 0, 0)),
        ],
        out_shape=[
            jax.ShapeDtypeStruct((H, N, hd), jnp.float32),
            jax.ShapeDtypeStruct((H, N, hd), jnp.float32),
            jax.ShapeDtypeStruct((H, N, hd), jnp.float32),
            jax.ShapeDtypeStruct((H, N, 1), jnp.int32),
        ],
    )(x2, Wq.reshape(H, hd, D), Wk.reshape(H, hd, D), Wv.reshape(H, hd, D),
      Wr.reshape(H, K, D))

    node_row = node.reshape(H, 1, N)

    attn = pl.pallas_call(
        _attn_kernel,
        grid=(H, N // QBLK),
        in_specs=[
            pl.BlockSpec((1, QBLK, hd), lambda h, i: (h, i, 0)),
            pl.BlockSpec((1, N, hd), lambda h, i: (h, 0, 0)),
            pl.BlockSpec((1, N, hd), lambda h, i: (h, 0, 0)),
            pl.BlockSpec((1, QBLK, 1), lambda h, i: (h, i, 0)),
            pl.BlockSpec((1, 1, N), lambda h, i: (h, 0, 0)),
        ],
        out_specs=pl.BlockSpec((1, QBLK, hd), lambda h, i: (h, i, 0)),
        out_shape=jax.ShapeDtypeStruct((H, N, hd), jnp.float32),
    )(q, k, v, node, node_row)

    wo_h = Wo.reshape(D, H, hd).transpose(1, 0, 2)
    out = pl.pallas_call(
        _outproj_kernel,
        grid=(H,),
        in_specs=[
            pl.BlockSpec((1, N, hd), lambda h: (h, 0, 0)),
            pl.BlockSpec((1, D, hd), lambda h: (h, 0, 0)),
        ],
        out_specs=pl.BlockSpec((N, D), lambda h: (0, 0)),
        out_shape=jax.ShapeDtypeStruct((N, D), jnp.float32),
    )(attn, wo_h)
    return out.reshape(B, N, D)
